# baseline (speedup 1.0000x reference)
import jax
import jax.numpy as jnp
from jax import lax
from jax.experimental import pallas as pl
from jax.experimental.pallas import tpu as pltpu
from jax.experimental.pallas import tpu_sc as plsc

S, B, T, D = 20, 1024, 26, 128
N = S * B * T
NW = 32
PER_W = N // NW
CH = 128
NCH = PER_W // CH


def _body(tags_hbm, probs_hbm, table_hbm, out_hbm,
          idx_all, prob_all, rows0, rows1, ob0, ob1,
          gsem0, gsem1, osem0, osem1):
    c = lax.axis_index("c")
    s = lax.axis_index("s")
    wid = s * 2 + c
    base = wid * PER_W

    rows = (rows0, rows1)
    obuf = (ob0, ob1)
    gsem = (gsem0, gsem1)
    osem = (osem0, osem1)

    pltpu.sync_copy(tags_hbm.at[pl.ds(base, PER_W)], idx_all)
    pltpu.sync_copy(probs_hbm.at[pl.ds(base, PER_W)], prob_all)

    def fire_gather(b, g):
        pltpu.async_copy(table_hbm.at[idx_all.at[pl.ds(g * CH, CH)]],
                         rows[b], gsem[b])

    def wait_gather(b):
        pltpu.make_async_copy(table_hbm.at[pl.ds(0, CH)], rows[b],
                              gsem[b]).wait()

    def fire_out(b, g):
        pltpu.async_copy(obuf[b], out_hbm.at[pl.ds(base + g * CH, CH)], osem[b])

    def wait_out(b):
        pltpu.make_async_copy(obuf[b], out_hbm.at[pl.ds(0, CH)], osem[b]).wait()

    def compute(b, g):
        def group(q, carry):
            pv = prob_all[pl.ds(g * CH + q * 16, 16)]
            for k in range(16):
                r = q * 16 + k
                pb = pv[k]
                for j in range(8):
                    sl = pl.ds(j * 16, 16)
                    obuf[b][r, sl] = rows[b][r, sl] * pb
            return carry
        lax.fori_loop(0, CH // 16, group, 0, unroll=4)

    fire_gather(0, 0)
    fire_gather(1, 1)

    for b in range(2):
        wait_gather(b)
        compute(b, b)
        fire_out(b, b)
        fire_gather(b, b + 2)

    def pair(qq, carry):
        for b in range(2):
            g = qq * 2 + b
            wait_gather(b)
            wait_out(b)
            compute(b, g)
            fire_out(b, g)
            gn = jnp.minimum(g + 2, NCH - 1)
            fire_gather(b, gn)
        return carry

    lax.fori_loop(1, NCH // 2, pair, 0)

    for b in range(2):
        wait_gather(b)
        wait_out(b)


@jax.jit
def _run(tags_flat, probs_flat, table):
    mesh = plsc.VectorSubcoreMesh(core_axis_name="c", subcore_axis_name="s")
    out = pl.kernel(
        _body,
        out_type=jax.ShapeDtypeStruct((N, D), jnp.float32),
        mesh=mesh,
        scratch_types=[
            pltpu.VMEM((PER_W,), jnp.int32),
            pltpu.VMEM((PER_W,), jnp.float32),
            pltpu.VMEM((CH, D), jnp.float32),
            pltpu.VMEM((CH, D), jnp.float32),
            pltpu.VMEM((CH, D), jnp.float32),
            pltpu.VMEM((CH, D), jnp.float32),
            pltpu.SemaphoreType.DMA,
            pltpu.SemaphoreType.DMA,
            pltpu.SemaphoreType.DMA,
            pltpu.SemaphoreType.DMA,
        ],
    )(tags_flat, probs_flat, table)
    return out


def kernel(tags, probs, table):
    tt = tags.transpose(0, 2, 1).reshape(N)
    pp = probs.transpose(0, 2, 1).reshape(N)
    out = _run(tt, pp, table)
    return out.reshape(S, T, B, D).transpose(0, 2, 1, 3)


# speedup vs baseline: 1.0337x; 1.0337x over previous
import jax
import jax.numpy as jnp
from jax import lax
from jax.experimental import pallas as pl
from jax.experimental.pallas import tpu as pltpu
from jax.experimental.pallas import tpu_sc as plsc

S, B, T, D = 20, 1024, 26, 128
N = S * B * T
NW = 32
PER_W = N // NW
CH = 128
NCH = PER_W // CH
NB = 3


def _body(tags_hbm, probs_hbm, table_hbm, out_hbm,
          idx_all, pbuf, rows0, rows1, rows2, ob0, ob1, ob2,
          gsem0, gsem1, gsem2, osem0, osem1, osem2):
    c = lax.axis_index("c")
    s = lax.axis_index("s")
    wid = s * 2 + c
    base = wid * PER_W

    rows = (rows0, rows1, rows2)
    obuf = (ob0, ob1, ob2)
    gsem = (gsem0, gsem1, gsem2)
    osem = (osem0, osem1, osem2)

    pltpu.sync_copy(tags_hbm.at[pl.ds(base, PER_W)], idx_all)

    def fire_gather(b, g):
        pltpu.async_copy(probs_hbm.at[pl.ds(base + g * CH, CH)],
                         pbuf.at[b], gsem[b])
        pltpu.async_copy(table_hbm.at[idx_all.at[pl.ds(g * CH, CH)]],
                         rows[b], gsem[b])

    def wait_gather(b):
        pltpu.make_async_copy(probs_hbm.at[pl.ds(0, CH)], pbuf.at[b],
                              gsem[b]).wait()
        pltpu.make_async_copy(table_hbm.at[pl.ds(0, CH)], rows[b],
                              gsem[b]).wait()

    def fire_out(b, g):
        pltpu.async_copy(obuf[b], out_hbm.at[pl.ds(base + g * CH, CH)], osem[b])

    def wait_out(b):
        pltpu.make_async_copy(obuf[b], out_hbm.at[pl.ds(0, CH)], osem[b]).wait()

    def compute(b):
        def group(q, carry):
            pv = pbuf[b, pl.ds(q * 16, 16)]
            for k in range(16):
                r = q * 16 + k
                pb = pv[k]
                for j in range(8):
                    sl = pl.ds(j * 16, 16)
                    obuf[b][r, sl] = rows[b][r, sl] * pb
            return carry
        lax.fori_loop(0, CH // 16, group, 0, unroll=2)

    for b in range(NB):
        fire_gather(b, b)

    for b in range(NB):
        wait_gather(b)
        compute(b)
        fire_out(b, b)
        fire_gather(b, b + NB)

    def triple(qq, carry):
        for b in range(NB):
            g = qq * NB + b
            wait_gather(b)
            wait_out(b)
            compute(b)
            fire_out(b, g)
            gn = jnp.minimum(g + NB, NCH - 1)
            fire_gather(b, gn)
        return carry

    lax.fori_loop(1, (NCH - 1) // NB, triple, 0)

    wait_gather(0)
    wait_out(0)
    compute(0)
    fire_out(0, NCH - 1)

    for b in range(1, NB):
        wait_gather(b)
        wait_out(b)
    wait_out(0)


@jax.jit
def _run(tags_flat, probs_flat, table):
    mesh = plsc.VectorSubcoreMesh(core_axis_name="c", subcore_axis_name="s")
    out = pl.kernel(
        _body,
        out_type=jax.ShapeDtypeStruct((N, D), jnp.float32),
        mesh=mesh,
        scratch_types=[
            pltpu.VMEM((PER_W,), jnp.int32),
            pltpu.VMEM((NB, CH), jnp.float32),
            pltpu.VMEM((CH, D), jnp.float32),
            pltpu.VMEM((CH, D), jnp.float32),
            pltpu.VMEM((CH, D), jnp.float32),
            pltpu.VMEM((CH, D), jnp.float32),
            pltpu.VMEM((CH, D), jnp.float32),
            pltpu.VMEM((CH, D), jnp.float32),
            pltpu.SemaphoreType.DMA,
            pltpu.SemaphoreType.DMA,
            pltpu.SemaphoreType.DMA,
            pltpu.SemaphoreType.DMA,
            pltpu.SemaphoreType.DMA,
            pltpu.SemaphoreType.DMA,
        ],
    )(tags_flat, probs_flat, table)
    return out


def kernel(tags, probs, table):
    tt = tags.transpose(0, 2, 1).reshape(N)
    pp = probs.transpose(0, 2, 1).reshape(N)
    out = _run(tt, pp, table)
    return out.reshape(S, T, B, D).transpose(0, 2, 1, 3)
